# baseline (speedup 1.0000x reference)
import jax
import jax.numpy as jnp
from jax import lax
from jax.experimental import pallas as pl
from jax.experimental.pallas import tpu as pltpu


_CH = 512
_NB = 6


def _poly_body(imp_ref, c_ref, x_hbm, o_hbm, abg, inb, outb, insem, outsem):
    rows = x_hbm.shape[0]
    nch = rows // _CH

    m = jax.nn.sigmoid(imp_ref[...])
    m = m / (jnp.mean(m) + 1e-6)
    m2 = m * m
    abg[0] = c_ref[0] * m
    abg[1] = c_ref[1] * m2
    abg[2] = c_ref[2] * (m2 * m)

    def in_copy(t, s):
        return pltpu.make_async_copy(
            x_hbm.at[pl.ds(t * _CH, _CH), :], inb.at[s], insem.at[s])

    def out_copy(t, s):
        return pltpu.make_async_copy(
            outb.at[s], o_hbm.at[pl.ds(t * _CH, _CH), :], outsem.at[s])

    for s in range(_NB):
        in_copy(s, s).start()

    def step(t, carry):
        s = lax.rem(t, _NB)
        in_copy(t, s).wait()

        @pl.when(t >= _NB)
        def _():
            out_copy(t - _NB, s).wait()

        a = abg[0]
        b = abg[1]
        g = abg[2]
        x = inb[s]
        outb[s] = x * (a + x * (b + x * g))
        out_copy(t, s).start()

        @pl.when(t + _NB < nch)
        def _():
            in_copy(t + _NB, s).start()

        return carry

    lax.fori_loop(0, nch, step, 0)

    for k in range(_NB):
        t = nch - _NB + k
        out_copy(t, lax.rem(t, _NB)).wait()


def kernel(x, coeffs, importance):
    B, T, D = x.shape
    rows = B * T
    x2 = x.reshape(rows, D)
    imp2 = importance.reshape(1, D)
    out = pl.pallas_call(
        _poly_body,
        in_specs=[
            pl.BlockSpec((1, D), lambda: (0, 0)),
            pl.BlockSpec(memory_space=pltpu.MemorySpace.SMEM),
            pl.BlockSpec(memory_space=pltpu.MemorySpace.HBM),
        ],
        out_specs=pl.BlockSpec(memory_space=pltpu.MemorySpace.HBM),
        out_shape=jax.ShapeDtypeStruct((rows, D), jnp.float32),
        scratch_shapes=[
            pltpu.VMEM((3, 1, D), jnp.float32),
            pltpu.VMEM((_NB, _CH, D), jnp.float32),
            pltpu.VMEM((_NB, _CH, D), jnp.float32),
            pltpu.SemaphoreType.DMA((_NB,)),
            pltpu.SemaphoreType.DMA((_NB,)),
        ],
    )(imp2, coeffs, x2)
    return out.reshape(B, T, D)


# speedup vs baseline: 1.0043x; 1.0043x over previous
import jax
import jax.numpy as jnp
from jax import lax
from jax.experimental import pallas as pl
from jax.experimental.pallas import tpu as pltpu


_CH = 1024
_NB = 3


def _poly_body(imp_ref, c_ref, x_hbm, o_hbm, abg, inb, outb, insem, outsem):
    rows = x_hbm.shape[0]
    nch = rows // _CH

    m = jax.nn.sigmoid(imp_ref[...])
    m = m / (jnp.mean(m) + 1e-6)
    m2 = m * m
    abg[0] = c_ref[0] * m
    abg[1] = c_ref[1] * m2
    abg[2] = c_ref[2] * (m2 * m)

    def in_copy(t, s):
        return pltpu.make_async_copy(
            x_hbm.at[pl.ds(t * _CH, _CH), :], inb.at[s], insem.at[s])

    def out_copy(t, s):
        return pltpu.make_async_copy(
            outb.at[s], o_hbm.at[pl.ds(t * _CH, _CH), :], outsem.at[s])

    for s in range(_NB):
        in_copy(s, s).start()

    def step(t, carry):
        s = lax.rem(t, _NB)
        in_copy(t, s).wait()

        @pl.when(t >= _NB)
        def _():
            out_copy(t - _NB, s).wait()

        a = abg[0]
        b = abg[1]
        g = abg[2]
        x = inb[s]
        outb[s] = x * (a + x * (b + x * g))
        out_copy(t, s).start()

        @pl.when(t + _NB < nch)
        def _():
            in_copy(t + _NB, s).start()

        return carry

    lax.fori_loop(0, nch, step, 0)

    for k in range(_NB):
        t = nch - _NB + k
        out_copy(t, lax.rem(t, _NB)).wait()


def kernel(x, coeffs, importance):
    B, T, D = x.shape
    rows = B * T
    x2 = x.reshape(rows, D)
    imp2 = importance.reshape(1, D)
    out = pl.pallas_call(
        _poly_body,
        in_specs=[
            pl.BlockSpec((1, D), lambda: (0, 0)),
            pl.BlockSpec(memory_space=pltpu.MemorySpace.SMEM),
            pl.BlockSpec(memory_space=pltpu.MemorySpace.HBM),
        ],
        out_specs=pl.BlockSpec(memory_space=pltpu.MemorySpace.HBM),
        out_shape=jax.ShapeDtypeStruct((rows, D), jnp.float32),
        scratch_shapes=[
            pltpu.VMEM((3, 1, D), jnp.float32),
            pltpu.VMEM((_NB, _CH, D), jnp.float32),
            pltpu.VMEM((_NB, _CH, D), jnp.float32),
            pltpu.SemaphoreType.DMA((_NB,)),
            pltpu.SemaphoreType.DMA((_NB,)),
        ],
    )(imp2, coeffs, x2)
    return out.reshape(B, T, D)
